# baseline (speedup 1.0000x reference)
import functools

import numpy as np
import jax
import jax.numpy as jnp
from jax.experimental import pallas as pl
from jax.experimental.pallas import tpu as pltpu

N = 1000
F = 64
L = 3
NRBF = 20
RCUT = 6.0

NP = 1024
BI = 128
BJ = 256
NI = NP // BI
NJ = NP // BJ

GAMMA = (NRBF / (RCUT - 0.5)) ** 2
CENTERS = np.linspace(0.5, RCUT, NRBF).astype(np.float32)
FAR = 1.0e6


def _dot(a, b):
    return jax.lax.dot_general(a, b, (((1,), (0,)), ((), ())),
                               preferred_element_type=jnp.float32)


def _bdot(a, b):
    return jax.lax.dot_general(a, b, (((2,), (1,)), ((0,), (0,))),
                               preferred_element_type=jnp.float32)


def _mp_layer_kernel(centers_ref, pos_ref, posT_ref, nsq_ref, nsqT_ref,
                     hT_ref, w1T_ref, b1T_ref, w2T_ref, b2T_ref,
                     w3T_ref, b3T_ref, w4T_ref, b4T_ref, outT_ref):
    ib = pl.program_id(0)
    i0 = ib * BI
    pos_i = pos_ref[pl.ds(i0, BI), :]
    nsq_i = nsq_ref[pl.ds(i0, BI), :]
    gi = i0 + jax.lax.broadcasted_iota(jnp.int32, (BI, BJ), 0)
    gj0 = jax.lax.broadcasted_iota(jnp.int32, (BI, BJ), 1)
    centers3 = centers_ref[:, :].reshape(1, NRBF, 1)
    w1b = jnp.broadcast_to(w1T_ref[:, :].reshape(1, F, NRBF), (BI, F, NRBF))
    w2b = jnp.broadcast_to(w2T_ref[:, :].reshape(1, F, F), (BI, F, F))
    b1_3 = b1T_ref[:, :].reshape(1, F, 1)
    b2_3 = b2T_ref[:, :].reshape(1, F, 1)

    macc = jnp.zeros((BI, F, BJ), jnp.float32)
    for jb in range(NJ):
        j0 = jb * BJ
        posT_j = posT_ref[:, pl.ds(j0, BJ)]
        gram = _dot(pos_i, posT_j)
        r2 = jnp.maximum(nsq_i + nsqT_ref[:, pl.ds(j0, BJ)] - 2.0 * gram, 0.0)
        r = jnp.sqrt(r2)
        gj = j0 + gj0
        ok = (gi != gj) & (gj < N)
        z = jnp.where(ok, r, FAR)
        cut = jnp.where(z < RCUT,
                        0.5 * (jnp.cos((np.pi / RCUT) * z) + 1.0), 0.0)
        z3 = z.reshape(BI, 1, BJ)
        cut3 = cut.reshape(BI, 1, BJ)
        rbf3 = jnp.exp(-GAMMA * (z3 - centers3) ** 2) * cut3
        t3 = jax.nn.silu(_bdot(w1b, rbf3) + b1_3)
        wm3 = _bdot(w2b, t3) + b2_3
        hjT = hT_ref[:, pl.ds(j0, BJ)]
        macc = macc + wm3 * hjT.reshape(1, F, BJ)

    agg = jnp.sum(macc, axis=2)
    aggT = agg.T
    d1 = jax.nn.silu(_dot(w3T_ref[:, :], aggT) + b3T_ref[:, :])
    deltaT = _dot(w4T_ref[:, :], d1) + b4T_ref[:, :]
    outT_ref[:, :] = hT_ref[:, pl.ds(i0, BI)] + deltaT


def _readout_kernel(hT_ref, maskT_ref, wo1T_ref, bo1T_ref, wo2T_ref,
                    bo2_ref, out_ref):
    t = jax.nn.silu(_dot(wo1T_ref[:, :], hT_ref[:, :]) + bo1T_ref[:, :])
    e = _dot(wo2T_ref[:, :], t) + bo2_ref[:, :]
    out_ref[:, :] = jnp.sum(e * maskT_ref[:, :]).reshape(1, 1)


def _full(shape):
    return pl.BlockSpec(shape, lambda i: tuple(0 for _ in shape))


@functools.partial(jax.jit, static_argnums=())
def kernel(positions, real_mask, emb, W1, b1, W2, b2, W3, b3, W4, b4,
           Wo1, bo1, Wo2, bo2):
    f32 = jnp.float32
    pos = jnp.pad(positions.astype(f32), ((0, NP - N), (0, 5)))
    posT = pos.T
    nsq = jnp.sum(pos * pos, axis=1, keepdims=True)
    nsqT = nsq.T
    maskT = jnp.pad(real_mask.astype(f32), (0, NP - N)).reshape(1, NP)
    hT = jnp.broadcast_to(emb.astype(f32).reshape(F, 1), (F, NP))

    layer_call = pl.pallas_call(
        _mp_layer_kernel,
        grid=(NI,),
        in_specs=[
            _full((NRBF, 1)), _full((NP, 8)), _full((8, NP)),
            _full((NP, 1)), _full((1, NP)), _full((F, NP)),
            _full((F, NRBF)), _full((F, 1)), _full((F, F)), _full((F, 1)),
            _full((F, F)), _full((F, 1)), _full((F, F)), _full((F, 1)),
        ],
        out_specs=pl.BlockSpec((F, BI), lambda i: (0, i)),
        out_shape=jax.ShapeDtypeStruct((F, NP), f32),
        compiler_params=pltpu.CompilerParams(
            dimension_semantics=("parallel",)),
    )

    centers_in = jnp.asarray(CENTERS).reshape(NRBF, 1)
    for l in range(L):
        hT = layer_call(centers_in, pos, posT, nsq, nsqT, hT,
                        W1[l].T, b1[l].reshape(F, 1),
                        W2[l].T, b2[l].reshape(F, 1),
                        W3[l].T, b3[l].reshape(F, 1),
                        W4[l].T, b4[l].reshape(F, 1))

    F2 = Wo1.shape[1]
    readout_call = pl.pallas_call(
        _readout_kernel,
        grid=(1,),
        in_specs=[_full((F, NP)), _full((1, NP)), _full((F2, F)),
                  _full((F2, 1)), _full((1, F2)), _full((1, 1))],
        out_specs=_full((1, 1)),
        out_shape=jax.ShapeDtypeStruct((1, 1), f32),
    )
    out = readout_call(hT, maskT, Wo1.T, bo1.reshape(F2, 1), Wo2.T,
                       bo2.reshape(1, 1))
    return out[0, 0]


# speedup vs baseline: 1.0575x; 1.0575x over previous
import functools

import numpy as np
import jax
import jax.numpy as jnp
from jax.experimental import pallas as pl
from jax.experimental.pallas import tpu as pltpu

N = 1000
F = 64
L = 3
NRBF = 20
RCUT = 6.0

NP = 1024
BI = 128
BJ = 512
NI = NP // BI
NJ = NP // BJ

GAMMA = (NRBF / (RCUT - 0.5)) ** 2
CENTERS = np.linspace(0.5, RCUT, NRBF).astype(np.float32)
FAR = 1.0e6


def _dot(a, b):
    return jax.lax.dot_general(a, b, (((1,), (0,)), ((), ())),
                               preferred_element_type=jnp.float32)


def _bdot(a, b):
    return jax.lax.dot_general(a, b, (((2,), (1,)), ((0,), (0,))),
                               preferred_element_type=jnp.float32)


def _mp_layer_kernel(centers_ref, pos_ref, posT_ref, nsq_ref, nsqT_ref,
                     hT_ref, w1T_ref, b1T_ref, w2T_ref, b2T_ref,
                     w3T_ref, b3T_ref, w4T_ref, b4T_ref, outT_ref):
    ib = pl.program_id(0)
    i0 = ib * BI
    pos_i = pos_ref[pl.ds(i0, BI), :]
    nsq_i = nsq_ref[pl.ds(i0, BI), :]
    gi = i0 + jax.lax.broadcasted_iota(jnp.int32, (BI, BJ), 0)
    gj0 = jax.lax.broadcasted_iota(jnp.int32, (BI, BJ), 1)
    centers3 = centers_ref[:, :].reshape(1, NRBF, 1)
    w1b = jnp.broadcast_to(w1T_ref[:, :].reshape(1, F, NRBF), (BI, F, NRBF))
    w2b = jnp.broadcast_to(w2T_ref[:, :].reshape(1, F, F), (BI, F, F))
    b1_3 = b1T_ref[:, :].reshape(1, F, 1)
    b2_3 = b2T_ref[:, :].reshape(1, F, 1)

    agg = jnp.zeros((BI, F), jnp.float32)
    for jb in range(NJ):
        j0 = jb * BJ
        posT_j = posT_ref[:, pl.ds(j0, BJ)]
        gram = _dot(pos_i, posT_j)
        r2 = jnp.maximum(nsq_i + nsqT_ref[:, pl.ds(j0, BJ)] - 2.0 * gram, 0.0)
        r = jnp.sqrt(r2)
        gj = j0 + gj0
        ok = (gi != gj) & (gj < N)
        z = jnp.where(ok, r, FAR)
        cut = jnp.where(z < RCUT,
                        0.5 * (jnp.cos((np.pi / RCUT) * z) + 1.0), 0.0)
        z3 = z.reshape(BI, 1, BJ)
        cut3 = cut.reshape(BI, 1, BJ)
        rbf3 = jnp.exp(-GAMMA * (z3 - centers3) ** 2) * cut3
        t3 = jax.nn.silu(_bdot(w1b, rbf3) + b1_3)
        wm3 = _bdot(w2b, t3) + b2_3
        hjT = hT_ref[:, pl.ds(j0, BJ)]
        msgs = wm3 * hjT.reshape(1, F, BJ)
        agg = agg + jnp.sum(msgs, axis=2)

    aggT = agg.T
    d1 = jax.nn.silu(_dot(w3T_ref[:, :], aggT) + b3T_ref[:, :])
    deltaT = _dot(w4T_ref[:, :], d1) + b4T_ref[:, :]
    outT_ref[:, :] = hT_ref[:, pl.ds(i0, BI)] + deltaT


def _readout_kernel(hT_ref, maskT_ref, wo1T_ref, bo1T_ref, wo2T_ref,
                    bo2_ref, out_ref):
    t = jax.nn.silu(_dot(wo1T_ref[:, :], hT_ref[:, :]) + bo1T_ref[:, :])
    e = _dot(wo2T_ref[:, :], t) + bo2_ref[:, :]
    out_ref[:, :] = jnp.sum(e * maskT_ref[:, :]).reshape(1, 1)


def _full(shape):
    return pl.BlockSpec(shape, lambda i: tuple(0 for _ in shape))


@functools.partial(jax.jit, static_argnums=())
def kernel(positions, real_mask, emb, W1, b1, W2, b2, W3, b3, W4, b4,
           Wo1, bo1, Wo2, bo2):
    f32 = jnp.float32
    pos = jnp.pad(positions.astype(f32), ((0, NP - N), (0, 5)))
    posT = pos.T
    nsq = jnp.sum(pos * pos, axis=1, keepdims=True)
    nsqT = nsq.T
    maskT = jnp.pad(real_mask.astype(f32), (0, NP - N)).reshape(1, NP)
    hT = jnp.broadcast_to(emb.astype(f32).reshape(F, 1), (F, NP))

    layer_call = pl.pallas_call(
        _mp_layer_kernel,
        grid=(NI,),
        in_specs=[
            _full((NRBF, 1)), _full((NP, 8)), _full((8, NP)),
            _full((NP, 1)), _full((1, NP)), _full((F, NP)),
            _full((F, NRBF)), _full((F, 1)), _full((F, F)), _full((F, 1)),
            _full((F, F)), _full((F, 1)), _full((F, F)), _full((F, 1)),
        ],
        out_specs=pl.BlockSpec((F, BI), lambda i: (0, i)),
        out_shape=jax.ShapeDtypeStruct((F, NP), f32),
        compiler_params=pltpu.CompilerParams(
            dimension_semantics=("parallel",)),
    )

    centers_in = jnp.asarray(CENTERS).reshape(NRBF, 1)
    for l in range(L):
        hT = layer_call(centers_in, pos, posT, nsq, nsqT, hT,
                        W1[l].T, b1[l].reshape(F, 1),
                        W2[l].T, b2[l].reshape(F, 1),
                        W3[l].T, b3[l].reshape(F, 1),
                        W4[l].T, b4[l].reshape(F, 1))

    F2 = Wo1.shape[1]
    readout_call = pl.pallas_call(
        _readout_kernel,
        grid=(1,),
        in_specs=[_full((F, NP)), _full((1, NP)), _full((F2, F)),
                  _full((F2, 1)), _full((1, F2)), _full((1, 1))],
        out_specs=_full((1, 1)),
        out_shape=jax.ShapeDtypeStruct((1, 1), f32),
    )
    out = readout_call(hT, maskT, Wo1.T, bo1.reshape(F2, 1), Wo2.T,
                       bo2.reshape(1, 1))
    return out[0, 0]
